# baseline (speedup 1.0000x reference)
import jax
import jax.numpy as jnp
from jax import lax
from jax.experimental import pallas as pl
from jax.experimental.pallas import tpu as pltpu
from jax.experimental.pallas import tpu_sc as plsc

VOCAB = 1_000_000
D = 64
BATCH = 4096
HIST = 200
NC, NS = 2, 16
NW = NC * NS
ROWS_PW = BATCH // NW
S0, S1 = 104, 96
NBUF = 2
NCHUNK = 2 * ROWS_PW
GROUPS = NCHUNK // NBUF

_SIZE = (S0, S1)
_OFF = (0, S0)

RA = 31248
CH = 248
NCH = RA // CH
LEFT = VOCAB - NW * RA


def _wid():
    return lax.axis_index("s") * NC + lax.axis_index("c")


def _detile_body(table_hbm, scr_hbm, bufT, bufO, isem, osem):
    wid = _wid()
    base = wid * RA

    def in_desc(i, b, rows=CH):
        return pltpu.make_async_copy(
            table_hbm.at[pl.ds(base + i * CH, rows), :],
            bufT.at[b, pl.ds(0, rows)],
            isem.at[b],
        )

    def out_desc(i, b, rows=CH):
        return pltpu.make_async_copy(
            bufO.at[b, pl.ds(0, rows)],
            scr_hbm.at[pl.ds(base + i * CH, rows), :],
            osem.at[b],
        )

    def widen(b, rows=CH):
        def row(r, carry):
            for c in range(D // 16):
                bufO[b, r, pl.ds(16 * c, 16)] = bufT[b, r, pl.ds(16 * c, 16)]
            return carry

        lax.fori_loop(0, rows, row, 0)

    in_desc(0, 0).start()
    in_desc(1, 1).start()

    def pair(pp, carry):
        for h in range(2):
            i = 2 * pp + h
            b = h
            in_desc(i, b).wait()

            @pl.when(i >= 2)
            def _():
                out_desc(i - 2, b).wait()

            widen(b)
            out_desc(i, b).start()

            @pl.when(i + 2 < NCH)
            def _():
                in_desc(i + 2, b).start()

        return carry

    lax.fori_loop(0, NCH // 2, pair, 0)
    out_desc(NCH - 2, 0).wait()
    out_desc(NCH - 1, 1).wait()

    @pl.when(wid == 0)
    def _():
        r0 = NW * RA
        left_in = pltpu.make_async_copy(
            table_hbm.at[pl.ds(r0, LEFT), :],
            bufT.at[0, pl.ds(0, LEFT)],
            isem.at[0],
        )
        left_in.start()
        left_in.wait()
        widen(0, LEFT)
        left_out = pltpu.make_async_copy(
            bufO.at[0, pl.ds(0, LEFT)],
            scr_hbm.at[pl.ds(r0, LEFT), :],
            osem.at[0],
        )
        left_out.start()
        left_out.wait()


def _gather_body(x_hbm, scr_hbm, out_hbm, idx_v, rows_v, gsem, osem):
    wid = _wid()
    b0 = wid * ROWS_PW
    pltpu.sync_copy(x_hbm.at[pl.ds(b0, ROWS_PW)], idx_v)

    def gather_desc(g, bank, b):
        j = g * (NBUF // 2) + (b >> 1)
        p = b & 1
        return pltpu.make_async_copy(
            scr_hbm.at[idx_v.at[j, pl.ds(_OFF[p], _SIZE[p])]],
            rows_v.at[bank, b, pl.ds(0, _SIZE[p])],
            gsem.at[bank, b],
        )

    def write_desc(g, bank, b):
        j = g * (NBUF // 2) + (b >> 1)
        p = b & 1
        row0 = (b0 + j) * HIST + _OFF[p]
        return pltpu.make_async_copy(
            rows_v.at[bank, b, pl.ds(0, _SIZE[p])],
            out_hbm.at[pl.ds(row0, _SIZE[p])],
            osem.at[bank, b],
        )

    for b in range(NBUF):
        gather_desc(0, 0, b).start()

    def pair(pp, carry):
        for h in range(2):
            g = 2 * pp + h
            bank = h
            for b in range(NBUF):

                @pl.when(g + 1 < GROUPS)
                def _():
                    @pl.when(g >= 1)
                    def _():
                        write_desc(g - 1, 1 - bank, b).wait()

                    gather_desc(g + 1, 1 - bank, b).start()

            for b in range(NBUF):
                gather_desc(g, bank, b).wait()
                write_desc(g, bank, b).start()
        return carry

    lax.fori_loop(0, GROUPS // 2, pair, 0)
    for b in range(NBUF):
        write_desc(GROUPS - 2, 0, b).wait()
        write_desc(GROUPS - 1, 1, b).wait()


@jax.jit
def kernel(x, table):
    mesh = plsc.VectorSubcoreMesh(
        core_axis_name="c", subcore_axis_name="s", num_cores=NC, num_subcores=NS
    )
    scratch128 = pl.kernel(
        _detile_body,
        out_type=jax.ShapeDtypeStruct((VOCAB, 2 * D), jnp.float32),
        mesh=mesh,
        scratch_types=[
            pltpu.VMEM((2, CH, D), jnp.float32),
            pltpu.VMEM((2, CH, 2 * D), jnp.float32),
            pltpu.SemaphoreType.DMA((2,)),
            pltpu.SemaphoreType.DMA((2,)),
        ],
    )(table)
    out128 = pl.kernel(
        _gather_body,
        out_type=jax.ShapeDtypeStruct((BATCH * HIST, 2 * D), jnp.float32),
        mesh=mesh,
        scratch_types=[
            pltpu.VMEM((ROWS_PW, HIST), jnp.int32),
            pltpu.VMEM((2, NBUF, S0, 2 * D), jnp.float32),
            pltpu.SemaphoreType.DMA((2, NBUF)),
            pltpu.SemaphoreType.DMA((2, NBUF)),
        ],
        compiler_params=pltpu.CompilerParams(use_tc_tiling_on_sc=False),
    )(x, scratch128)
    return out128[:, :D].reshape(BATCH, HIST, D)


# speedup vs baseline: 1.1608x; 1.1608x over previous
import jax
import jax.numpy as jnp
from jax import lax
from jax.experimental import pallas as pl
from jax.experimental.pallas import tpu as pltpu
from jax.experimental.pallas import tpu_sc as plsc

VOCAB = 1_000_000
D = 64
BATCH = 4096
HIST = 200
NC, NS = 2, 16
NW = NC * NS
ROWS_PW = BATCH // NW
S0, S1 = 104, 96
NBUF = 2
NCHUNK = 2 * ROWS_PW
GROUPS = NCHUNK // NBUF

_SIZE = (S0, S1)
_OFF = (0, S0)


def _gather_body(x_hbm, scr_hbm, out_hbm, idx_v, rows_v, gsem, osem):
    wid = lax.axis_index("s") * NC + lax.axis_index("c")
    b0 = wid * ROWS_PW
    pltpu.sync_copy(x_hbm.at[pl.ds(b0, ROWS_PW)], idx_v)

    def gather_desc(g, bank, b):
        j = g * (NBUF // 2) + (b >> 1)
        p = b & 1
        return pltpu.make_async_copy(
            scr_hbm.at[idx_v.at[j, pl.ds(_OFF[p], _SIZE[p])]],
            rows_v.at[bank, b, pl.ds(0, _SIZE[p])],
            gsem.at[bank, b],
        )

    def write_desc(g, bank, b):
        j = g * (NBUF // 2) + (b >> 1)
        p = b & 1
        row0 = (b0 + j) * HIST + _OFF[p]
        return pltpu.make_async_copy(
            rows_v.at[bank, b, pl.ds(0, _SIZE[p])],
            out_hbm.at[pl.ds(row0, _SIZE[p])],
            osem.at[bank, b],
        )

    for b in range(NBUF):
        gather_desc(0, 0, b).start()

    def pair(pp, carry):
        for h in range(2):
            g = 2 * pp + h
            bank = h
            for b in range(NBUF):

                @pl.when(g + 1 < GROUPS)
                def _():
                    @pl.when(g >= 1)
                    def _():
                        write_desc(g - 1, 1 - bank, b).wait()

                    gather_desc(g + 1, 1 - bank, b).start()

            for b in range(NBUF):
                gather_desc(g, bank, b).wait()
                write_desc(g, bank, b).start()
        return carry

    lax.fori_loop(0, GROUPS // 2, pair, 0)
    for b in range(NBUF):
        write_desc(GROUPS - 2, 0, b).wait()
        write_desc(GROUPS - 1, 1, b).wait()


@jax.jit
def kernel(x, table):
    table128 = jnp.pad(table, ((0, 0), (0, D)))
    mesh = plsc.VectorSubcoreMesh(
        core_axis_name="c", subcore_axis_name="s", num_cores=NC, num_subcores=NS
    )
    out128 = pl.kernel(
        _gather_body,
        out_type=jax.ShapeDtypeStruct((BATCH * HIST, 2 * D), jnp.float32),
        mesh=mesh,
        scratch_types=[
            pltpu.VMEM((ROWS_PW, HIST), jnp.int32),
            pltpu.VMEM((2, NBUF, S0, 2 * D), jnp.float32),
            pltpu.SemaphoreType.DMA((2, NBUF)),
            pltpu.SemaphoreType.DMA((2, NBUF)),
        ],
        compiler_params=pltpu.CompilerParams(use_tc_tiling_on_sc=False),
    )(x, table128)
    return out128[:, :D].reshape(BATCH, HIST, D)


# speedup vs baseline: 1.2675x; 1.0919x over previous
import jax
import jax.numpy as jnp
from jax import lax
from jax.experimental import pallas as pl
from jax.experimental.pallas import tpu as pltpu
from jax.experimental.pallas import tpu_sc as plsc

VOCAB = 1_000_000
D = 64
BATCH = 4096
HIST = 200
NC, NS = 2, 16
NW = NC * NS
ROWS_PW = BATCH // NW
S0, S1 = 104, 96
NBUF = 4
NCHUNK = 2 * ROWS_PW
GROUPS = NCHUNK // NBUF

_SIZE = (S0, S1)
_OFF = (0, S0)


def _body(x_hbm, table_hbm, out_hbm, idx_v, rows_v, gsem, osem):
    c = lax.axis_index("c")
    s = lax.axis_index("s")
    wid = s * NC + c
    b0 = wid * ROWS_PW
    pltpu.sync_copy(x_hbm.at[pl.ds(b0, ROWS_PW)], idx_v)

    def gather_desc(g, bank, b):
        j = g * (NBUF // 2) + (b >> 1)
        p = b & 1
        return pltpu.make_async_copy(
            table_hbm.at[idx_v.at[j, pl.ds(_OFF[p], _SIZE[p])]],
            rows_v.at[bank, b, pl.ds(0, _SIZE[p])],
            gsem.at[bank, b],
        )

    def write_desc(g, bank, b):
        j = g * (NBUF // 2) + (b >> 1)
        p = b & 1
        row0 = (b0 + j) * HIST + _OFF[p]
        return pltpu.make_async_copy(
            rows_v.at[bank, b, pl.ds(0, _SIZE[p])],
            out_hbm.at[pl.ds(row0, _SIZE[p]), pl.ds(0, D)],
            osem.at[bank, b],
        )

    for b in range(NBUF):
        gather_desc(0, 0, b).start()

    def pair(pp, carry):
        for h in range(2):
            g = 2 * pp + h
            bank = h
            for b in range(NBUF):

                @pl.when(g + 1 < GROUPS)
                def _():
                    @pl.when(g >= 1)
                    def _():
                        write_desc(g - 1, 1 - bank, b).wait()

                    gather_desc(g + 1, 1 - bank, b).start()

            for b in range(NBUF):
                gather_desc(g, bank, b).wait()
                write_desc(g, bank, b).start()
        return carry

    lax.fori_loop(0, GROUPS // 2, pair, 0)
    for b in range(NBUF):
        write_desc(GROUPS - 2, 0, b).wait()
        write_desc(GROUPS - 1, 1, b).wait()


@jax.jit
def kernel(x, table):
    table_lin = table.reshape(VOCAB // 2, 2 * D).reshape(VOCAB, D)
    mesh = plsc.VectorSubcoreMesh(
        core_axis_name="c", subcore_axis_name="s", num_cores=NC, num_subcores=NS
    )
    out128 = pl.kernel(
        _body,
        out_type=jax.ShapeDtypeStruct((BATCH * HIST, 2 * D), jnp.float32),
        mesh=mesh,
        scratch_types=[
            pltpu.VMEM((ROWS_PW, HIST), jnp.int32),
            pltpu.VMEM((2, NBUF, S0, D), jnp.float32),
            pltpu.SemaphoreType.DMA((2, NBUF)),
            pltpu.SemaphoreType.DMA((2, NBUF)),
        ],
        compiler_params=pltpu.CompilerParams(use_tc_tiling_on_sc=False),
    )(x, table_lin)
    return out128[:, :D].reshape(BATCH, HIST, D)
